# baseline (speedup 1.0000x reference)
import functools

import jax
import jax.numpy as jnp
from jax import lax
from jax.experimental import pallas as pl
from jax.experimental.pallas import tpu as pltpu
from jax.experimental.pallas import tpu_sc as plsc

B, N, K = 8, 2048, 20
M = B * N * K
NC, NS = 2, 16
NW = NC * NS
LRELU = 0.2
TR = 256
TN = 128


def _knn_body(xr_ref, xa_ref, out_ref):
    b = pl.program_id(0)
    xr = xr_ref[0]
    xa = xa_ref[0]
    g = lax.dot_general(xr, xa, (((1,), (1,)), ((), ())),
                        preferred_element_type=jnp.float32)
    xxr = jnp.sum(xr * xr, axis=1, keepdims=True)
    ones = jnp.ones((1, xa.shape[1]), jnp.float32)
    xxa = lax.dot_general(ones, xa * xa, (((1,), (1,)), ((), ())),
                          preferred_element_type=jnp.float32)
    pd = 2.0 * g - xxr - xxa
    iota_m = lax.broadcasted_iota(jnp.int32, pd.shape, 1)
    iota_k = lax.broadcasted_iota(jnp.int32, (TR, K), 1)
    acc = jnp.zeros((TR, K), jnp.int32)
    base = b * N
    for t in range(K):
        m = jnp.max(pd, axis=1, keepdims=True)
        am = jnp.min(jnp.where(pd == m, iota_m, N), axis=1, keepdims=True)
        acc = jnp.where(iota_k == t, am + base, acc)
        pd = jnp.where(iota_m == am, -jnp.inf, pd)
    out_ref[0] = acc


def _knn(xf):
    C = xf.shape[-1]
    return pl.pallas_call(
        _knn_body,
        grid=(B, N // TR),
        in_specs=[
            pl.BlockSpec((1, TR, C), lambda b, r: (b, r, 0)),
            pl.BlockSpec((1, N, C), lambda b, r: (b, 0, 0)),
        ],
        out_specs=pl.BlockSpec((1, TR, K), lambda b, r: (b, r, 0)),
        out_shape=jax.ShapeDtypeStruct((B, N, K), jnp.int32),
    )(xf, xf)


def _make_sc_gather(C):
    b_per_w = M // NW
    win = 1280
    nwin = b_per_w // win
    mesh = plsc.VectorSubcoreMesh(core_axis_name="c", subcore_axis_name="s",
                                  num_cores=NC, num_subcores=NS)

    @functools.partial(
        pl.kernel, mesh=mesh,
        compiler_params=pltpu.CompilerParams(use_tc_tiling_on_sc=False),
        out_type=jax.ShapeDtypeStruct((M, C), jnp.float32),
        scratch_types=[
            pltpu.VMEM((win,), jnp.int32),
            pltpu.VMEM((win, C), jnp.float32),
            pltpu.SemaphoreType.DMA,
        ],
    )
    def k(table_hbm, idx_hbm, out_hbm, idx_v, rows_v, sem):
        wid = lax.axis_index("s") * NC + lax.axis_index("c")
        for w in range(nwin):
            base = wid * b_per_w + w * win
            pltpu.sync_copy(idx_hbm.at[pl.ds(base, win)], idx_v)
            pltpu.async_copy(table_hbm.at[idx_v], rows_v, sem).wait()
            pltpu.sync_copy(rows_v, out_hbm.at[pl.ds(base, win)])

    return k


_gather_cache = {}


def _gather(table, fidx):
    c = table.shape[-1]
    if c not in _gather_cache:
        _gather_cache[c] = _make_sc_gather(c)
    return _gather_cache[c](table, fidx)


def _edge_features(part_cfg, g_refs, x_refs):
    parts = []
    for cv, g_ref, x_ref in zip(part_cfg, g_refs, x_refs):
        g = g_ref[...]
        x = x_ref[0]
        g3 = g.reshape(TN, K, g.shape[-1])[:, :, :cv]
        x3 = x[:, :cv]
        d = (g3 - x3[:, None, :]).reshape(TN * K, cv)
        xr = jnp.broadcast_to(x3[:, None, :], (TN, K, cv)).reshape(TN * K, cv)
        parts.extend([d, xr])
    return jnp.concatenate(parts, axis=1)


def _make_stats(part_cfg, cps, folds):
    npart = len(part_cfg)
    ctot = sum(2 * c for c in part_cfg)
    nsteps = B * (N // TN)
    cnt = float(M)

    def body(*refs):
        g_refs = refs[0:2 * npart:2]
        x_refs = refs[1:2 * npart:2]
        w_refs = refs[2 * npart:2 * npart + 3 * len(folds)]
        out_refs = refs[2 * npart + 3 * len(folds):-2]
        sum_scr, m2_scr = refs[-2], refs[-1]
        step = pl.program_id(0) * pl.num_programs(1) + pl.program_id(1)

        e = _edge_features(part_cfg, g_refs, x_refs)

        @pl.when(step == 0)
        def _():
            sum_scr[...] = jnp.zeros_like(sum_scr)
            m2_scr[...] = jnp.zeros_like(m2_scr)

        sum_scr[...] += jnp.sum(e, axis=0, keepdims=True)
        m2_scr[...] += lax.dot_general(e, e, (((0,), (0,)), ((), ())),
                                       preferred_element_type=jnp.float32)

        @pl.when(step == nsteps - 1)
        def _():
            mean_e = sum_scr[...] / cnt
            m2n = m2_scr[...] / cnt
            for f, (lo, hi, cout) in enumerate(folds):
                v = w_refs[3 * f][...]
                gr = w_refs[3 * f + 1][...]
                br = w_refs[3 * f + 2][...]
                me = mean_e[:, lo:hi]
                m2s = m2n[lo:hi, lo:hi]
                mean_y = lax.dot_general(me, v, (((1,), (0,)), ((), ())),
                                         preferred_element_type=jnp.float32)
                t = lax.dot_general(m2s, v, (((1,), (0,)), ((), ())),
                                    preferred_element_type=jnp.float32)
                ey2 = jnp.sum(t * v, axis=0, keepdims=True)
                var = ey2 - mean_y * mean_y
                scale = gr * lax.rsqrt(var + 1e-5)
                out_refs[2 * f][...] = v * scale
                out_refs[2 * f + 1][...] = br - mean_y * scale

    in_specs = []
    for cp in cps:
        in_specs.append(pl.BlockSpec((TN * K, cp), lambda b, r: (b * (N // TN) + r, 0)))
        in_specs.append(pl.BlockSpec((1, TN, cp), lambda b, r: (b, r, 0)))
    out_shapes, out_specs = [], []
    for (lo, hi, cout) in folds:
        in_specs.append(pl.BlockSpec((hi - lo, cout), lambda b, r: (0, 0)))
        in_specs.append(pl.BlockSpec((1, cout), lambda b, r: (0, 0)))
        in_specs.append(pl.BlockSpec((1, cout), lambda b, r: (0, 0)))
        out_shapes.append(jax.ShapeDtypeStruct((hi - lo, cout), jnp.float32))
        out_shapes.append(jax.ShapeDtypeStruct((1, cout), jnp.float32))
        out_specs.append(pl.BlockSpec((hi - lo, cout), lambda b, r: (0, 0)))
        out_specs.append(pl.BlockSpec((1, cout), lambda b, r: (0, 0)))

    return pl.pallas_call(
        body,
        grid=(B, N // TN),
        in_specs=in_specs,
        out_specs=out_specs,
        out_shape=out_shapes,
        scratch_shapes=[
            pltpu.VMEM((1, ctot), jnp.float32),
            pltpu.VMEM((ctot, ctot), jnp.float32),
        ],
    )


def _make_apply(part_cfg, cps, convs):
    npart = len(part_cfg)

    def body(*refs):
        g_refs = refs[0:2 * npart:2]
        x_refs = refs[1:2 * npart:2]
        w_refs = refs[2 * npart:2 * npart + 2 * len(convs)]
        out_refs = refs[2 * npart + 2 * len(convs):]
        e = _edge_features(part_cfg, g_refs, x_refs)
        for f, (lo, hi, cout) in enumerate(convs):
            wp = w_refs[2 * f][...]
            bp = w_refs[2 * f + 1][...]
            y = lax.dot_general(e[:, lo:hi], wp, (((1,), (0,)), ((), ())),
                                preferred_element_type=jnp.float32) + bp
            y = jnp.where(y >= 0.0, y, LRELU * y)
            out_refs[f][0] = jnp.max(y.reshape(TN, K, cout), axis=1)

    in_specs = []
    for cp in cps:
        in_specs.append(pl.BlockSpec((TN * K, cp), lambda b, r: (b * (N // TN) + r, 0)))
        in_specs.append(pl.BlockSpec((1, TN, cp), lambda b, r: (b, r, 0)))
    out_shapes, out_specs = [], []
    for (lo, hi, cout) in convs:
        in_specs.append(pl.BlockSpec((hi - lo, cout), lambda b, r: (0, 0)))
        in_specs.append(pl.BlockSpec((1, cout), lambda b, r: (0, 0)))
        out_shapes.append(jax.ShapeDtypeStruct((B, N, cout), jnp.float32))
        out_specs.append(pl.BlockSpec((1, TN, cout), lambda b, r: (b, r, 0)))

    return pl.pallas_call(
        body,
        grid=(B, N // TN),
        in_specs=in_specs,
        out_specs=out_specs,
        out_shape=out_shapes,
    )


def _head_body(a_ref, c_ref, l1_ref, b1_ref, l2_ref, b2_ref, l3_ref, b3_ref,
               o_ref):
    s = jnp.sum(a_ref[...], axis=1) + jnp.sum(c_ref[...], axis=1)
    rowi = lax.broadcasted_iota(jnp.int32, (2048, 64), 0)
    colj = lax.broadcasted_iota(jnp.int32, (2048, 64), 1)
    p = jnp.where(rowi // 32 == colj, 1.0, 0.0)
    m = lax.dot_general(s, p, (((1,), (0,)), ((), ())),
                        preferred_element_type=jnp.float32) / 2048.0
    h = m @ l1_ref[...] + b1_ref[...]
    h = jnp.where(h >= 0.0, h, LRELU * h)
    h = h @ l2_ref[...] + b2_ref[...]
    h = jnp.where(h >= 0.0, h, LRELU * h)
    h = h @ l3_ref[...] + b3_ref[...]
    o_ref[...] = jnp.where(h >= 0.0, h, LRELU * h)


def _head(xd1, xd2, L1w, L1b, L2w, L2b, L3w, L3b):
    a = xd1.reshape(B, 32, 2048)
    c = xd2.reshape(B, 32, 2048)
    return pl.pallas_call(
        _head_body,
        out_shape=jax.ShapeDtypeStruct((B, 11), jnp.float32),
    )(a, c, L1w, L1b.reshape(1, 64), L2w, L2b.reshape(1, 32),
      L3w, L3b.reshape(1, 11))


def kernel(x, W1, g1, b1, W2, g2, b2, Wd1, gd1, bd1, Wd2, gd2, bd2,
           L1w, L1b, L2w, L2b, L3w, L3b):
    xt3 = jnp.transpose(jnp.squeeze(x, 0), (1, 2, 0))
    xt3p = jnp.pad(xt3, ((0, 0), (0, 0), (0, 13)))

    fidx1 = _knn(xt3p).reshape(M)
    gth1 = _gather(xt3p.reshape(B * N, 16), fidx1)

    stats1 = _make_stats([3], [16], [(0, 6, 16)])
    wp1, bp1 = stats1(gth1, xt3p, W1.T, g1.reshape(1, 16), b1.reshape(1, 16))
    apply1 = _make_apply([3], [16], [(0, 6, 16)])
    (x1m,) = apply1(gth1, xt3p, wp1, bp1)

    fidx2 = _knn(x1m).reshape(M)
    gth2 = _gather(x1m.reshape(B * N, 16), fidx2)

    stats2 = _make_stats([3, 16], [16, 16], [(0, 38, 32), (6, 38, 32)])
    wp2, bp2, wpd1, bpd1 = stats2(
        gth1, xt3p, gth2, x1m,
        W2.T, g2.reshape(1, 32), b2.reshape(1, 32),
        Wd1.T, gd1.reshape(1, 32), bd1.reshape(1, 32))
    apply2 = _make_apply([3, 16], [16, 16], [(0, 38, 32), (6, 38, 32)])
    x2m, xd1 = apply2(gth1, xt3p, gth2, x1m, wp2, bp2, wpd1, bpd1)

    fidx3 = _knn(x2m).reshape(M)
    gth3 = _gather(x2m.reshape(B * N, 32), fidx3)

    stats3 = _make_stats([32], [32], [(0, 64, 32)])
    wpd2, bpd2 = stats3(gth3, x2m, Wd2.T, gd2.reshape(1, 32),
                        bd2.reshape(1, 32))
    apply3 = _make_apply([32], [32], [(0, 64, 32)])
    (xd2,) = apply3(gth3, x2m, wpd2, bpd2)

    return _head(xd1, xd2, L1w, L1b, L2w, L2b, L3w, L3b)
